# baseline (speedup 1.0000x reference)
import functools

import jax
import jax.numpy as jnp
from jax import lax
from jax.experimental import pallas as pl
from jax.experimental.pallas import tpu as pltpu
from jax.experimental.pallas import tpu_sc as plsc

VOCAB = 1000000
EMBED = 64
NTOKENS = 200
BATCH = 4096

TOTAL = BATCH * NTOKENS
NUM_WORKERS = 32
PER_WORKER = TOTAL // NUM_WORKERS
CHUNK = NTOKENS
NCHUNKS = PER_WORKER // CHUNK
NGRP = CHUNK // 16
TAIL = CHUNK - NGRP * 16
IDXPAD = CHUNK + 8

_mesh = plsc.VectorSubcoreMesh(core_axis_name="c", subcore_axis_name="s")


@functools.partial(
    pl.kernel,
    mesh=_mesh,
    out_type=jax.ShapeDtypeStruct((TOTAL, EMBED), jnp.float32),
    scratch_types=[
        pltpu.VMEM((IDXPAD,), jnp.int32),
        pltpu.VMEM((IDXPAD,), jnp.int32),
        pltpu.VMEM((IDXPAD,), jnp.int32),
        pltpu.VMEM((IDXPAD,), jnp.int32),
        pltpu.VMEM((IDXPAD,), jnp.int32),
        pltpu.VMEM((IDXPAD,), jnp.int32),
        pltpu.VMEM((IDXPAD, 128), jnp.float32),
        pltpu.VMEM((IDXPAD, 128), jnp.float32),
        pltpu.VMEM((CHUNK, EMBED), jnp.float32),
        pltpu.VMEM((CHUNK, EMBED), jnp.float32),
        pltpu.VMEM((CHUNK * EMBED,), jnp.float32),
        pltpu.SemaphoreType.DMA,
        pltpu.SemaphoreType.DMA,
        pltpu.SemaphoreType.DMA,
        pltpu.SemaphoreType.DMA,
        pltpu.SemaphoreType.DMA,
        pltpu.SemaphoreType.DMA,
    ],
    compiler_params=pltpu.CompilerParams(use_tc_tiling_on_sc=True),
)
def _embed_sc(tokens_hbm, table2_hbm, pos_hbm, out_hbm,
              idx_a, idx_b, hidx_a, hidx_b, hoff_a, hoff_b,
              rows_a, rows_b, res_a, res_b, pos_v,
              sem_ia, sem_ib, sem_ra, sem_rb, sem_oa, sem_ob):
    wid = lax.axis_index("s") * 2 + lax.axis_index("c")
    base = wid * PER_WORKER
    last = NCHUNKS - 1
    iota = lax.iota(jnp.int32, 16)

    pltpu.sync_copy(pos_hbm, pos_v)

    def idx_fetch(c, idx_v, sem):
        c = jnp.minimum(c, last)
        pltpu.async_copy(tokens_hbm.at[pl.ds(base + c * CHUNK, CHUNK)],
                         idx_v.at[pl.ds(0, CHUNK)], sem)

    def idx_wait(idx_v, sem):
        pltpu.make_async_copy(tokens_hbm.at[pl.ds(0, CHUNK)],
                              idx_v.at[pl.ds(0, CHUNK)], sem).wait()

    def fire_rows(idx_v, hidx_v, hoff_v, rows_v, sem):
        for g in range(NGRP + 1):
            tv = idx_v[pl.ds(g * 16, 16)]
            hidx_v[pl.ds(g * 16, 16)] = lax.shift_right_logical(tv, 1)
            hoff_v[pl.ds(g * 16, 16)] = lax.shift_left(
                lax.bitwise_and(tv, 1), 6)
        pltpu.async_copy(table2_hbm.at[hidx_v.at[pl.ds(0, 96)]],
                         rows_v.at[pl.ds(0, 96)], sem)
        pltpu.async_copy(table2_hbm.at[hidx_v.at[pl.ds(96, 104)]],
                         rows_v.at[pl.ds(96, 104)], sem)

    def drain_rows(rows_v, hidx_v, sem):
        pltpu.make_async_copy(table2_hbm.at[hidx_v.at[pl.ds(0, 96)]],
                              rows_v.at[pl.ds(0, 96)], sem).wait()
        pltpu.make_async_copy(table2_hbm.at[hidx_v.at[pl.ds(96, 104)]],
                              rows_v.at[pl.ds(96, 104)], sem).wait()

    def select_add(rows_v, hoff_v, res_v):
        def tok16(jg, carry):
            hv = hoff_v[pl.ds(jg * 16, 16)]
            for i in range(16):
                ho = hv[i]
                j = jg * 16 + i
                for c in range(4):
                    vals = rows_v[j, pl.ds(ho + c * 16, 16)]
                    pv = pos_v[pl.ds(j * EMBED + c * 16, 16)]
                    res_v[j, pl.ds(c * 16, 16)] = vals + pv
            return carry

        lax.fori_loop(0, NGRP, tok16, 0)
        hv = hoff_v[pl.ds(NGRP * 16, 16)]
        for i in range(TAIL):
            ho = hv[i]
            j = NGRP * 16 + i
            for c in range(4):
                vals = rows_v[j, pl.ds(ho + c * 16, 16)]
                pv = pos_v[pl.ds(j * EMBED + c * 16, 16)]
                res_v[j, pl.ds(c * 16, 16)] = vals + pv

    def out_start(res_v, c, sem):
        pltpu.async_copy(res_v, out_hbm.at[pl.ds(base + c * CHUNK, CHUNK)], sem)

    def out_wait(res_v, sem):
        pltpu.make_async_copy(res_v, out_hbm.at[pl.ds(0, CHUNK)], sem).wait()

    pltpu.sync_copy(tokens_hbm.at[pl.ds(base, CHUNK)], idx_a.at[pl.ds(0, CHUNK)])
    fire_rows(idx_a, hidx_a, hoff_a, rows_a, sem_ra)
    idx_fetch(1, idx_b, sem_ib)

    def pair_body(g, carry):
        ca = 2 * g

        idx_wait(idx_b, sem_ib)
        fire_rows(idx_b, hidx_b, hoff_b, rows_b, sem_rb)
        idx_fetch(ca + 2, idx_a, sem_ia)
        drain_rows(rows_a, hidx_a, sem_ra)

        @pl.when(g > 0)
        def _():
            out_wait(res_a, sem_oa)

        select_add(rows_a, hoff_a, res_a)
        out_start(res_a, ca, sem_oa)

        idx_wait(idx_a, sem_ia)
        fire_rows(idx_a, hidx_a, hoff_a, rows_a, sem_ra)
        idx_fetch(ca + 3, idx_b, sem_ib)
        drain_rows(rows_b, hidx_b, sem_rb)

        @pl.when(g > 0)
        def _():
            out_wait(res_b, sem_ob)

        select_add(rows_b, hoff_b, res_b)
        out_start(res_b, ca + 1, sem_ob)
        return carry

    lax.fori_loop(0, NCHUNKS // 2, pair_body, 0)

    idx_wait(idx_b, sem_ib)
    drain_rows(rows_a, hidx_a, sem_ra)
    out_wait(res_a, sem_oa)
    out_wait(res_b, sem_ob)


def kernel(tokens, input_embedding, position_embedding):
    flat = tokens.reshape(-1).astype(jnp.int32)
    table2 = input_embedding.reshape(VOCAB // 2, 2 * EMBED)
    pos_flat = position_embedding.reshape(-1)
    out = _embed_sc(flat, table2, pos_flat)
    return out.reshape(BATCH, NTOKENS, EMBED)
